# baseline (speedup 1.0000x reference)
import jax
import jax.numpy as jnp
from jax import lax
from jax.experimental import pallas as pl
from jax.experimental.pallas import tpu as pltpu
from jax.experimental.pallas import tpu_sc as plsc

_NC = 2
_NS = 16
_NW = _NC * _NS
_CH = 16
_RING = 6


def _sc_body(x_hbm, t_hbm, o_hbm, t_buf, *rest):
    ios = rest[:_RING]
    sin = rest[_RING : 2 * _RING]
    sout = rest[2 * _RING : 3 * _RING]
    D = 1024
    S = t_hbm.shape[0] // D
    B = x_hbm.shape[0] // (S * D)
    wrows = S // _NW
    wid = lax.axis_index("s") * _NC + lax.axis_index("c")
    nwords = _CH * D
    nchunks = wrows // _CH
    nunits = nchunks * B

    def unit_off(u):
        c, b = divmod(u, B)
        return b * (S * D) + (wid * wrows + c * _CH) * D

    def issue_in(u):
        return pltpu.async_copy(
            x_hbm.at[pl.ds(unit_off(u), nwords)], ios[u % _RING], sin[u % _RING]
        )

    def issue_out(u):
        return pltpu.async_copy(
            ios[u % _RING], o_hbm.at[pl.ds(unit_off(u), nwords)], sout[u % _RING]
        )

    pending_out = {}
    pending_in = {u: issue_in(u) for u in range(min(_RING, nunits))}
    for c in range(nchunks):
        pltpu.sync_copy(t_hbm.at[pl.ds((wid * wrows + c * _CH) * D, nwords)], t_buf)
        for b in range(B):
            u = c * B + b
            pending_in.pop(u).wait()
            io = ios[u % _RING]

            @plsc.parallel_loop(0, nwords, step=16, unroll=8)
            def _add(k):
                io[pl.ds(k, 16)] = io[pl.ds(k, 16)] + t_buf[pl.ds(k, 16)]

            pending_out[u] = issue_out(u)
            w = u - (_RING - 1)
            if w >= 0 and w + _RING < nunits:
                pending_out.pop(w).wait()
                pending_in[w + _RING] = issue_in(w + _RING)
    for u in sorted(pending_out):
        pending_out.pop(u).wait()


def kernel(inputs, pos_embedding_table):
    B, S, D = inputs.shape
    x = inputs.reshape(B * S * D)
    t = pos_embedding_table.reshape(S * D)
    mesh = plsc.VectorSubcoreMesh(core_axis_name="c", subcore_axis_name="s")
    out = pl.kernel(
        _sc_body,
        out_type=jax.ShapeDtypeStruct((B * S * D,), inputs.dtype),
        mesh=mesh,
        scratch_types=(
            [pltpu.VMEM((_CH * D,), jnp.float32)] * (1 + _RING)
            + [pltpu.SemaphoreType.DMA] * (2 * _RING)
        ),
    )(x, t)
    return out.reshape(B, S, D)


# speedup vs baseline: 1.2542x; 1.2542x over previous
import jax
import jax.numpy as jnp
from jax import lax
from jax.experimental import pallas as pl
from jax.experimental.pallas import tpu as pltpu
from jax.experimental.pallas import tpu_sc as plsc

_NC = 2
_NS = 16
_NW = _NC * _NS
_CH = 32


def _sc_body(x_hbm, t_hbm, o_hbm, t_buf, io0, io1, si0, si1, so0, so1):
    D = 1024
    S = t_hbm.shape[0] // D
    B_all = x_hbm.shape[0] // (S * D)
    xbase = (B_all - 1) * S * D
    wrows = S // _NW
    wid = lax.axis_index("s") * _NC + lax.axis_index("c")
    nwords = _CH * D
    nunits = wrows // _CH
    ios = (io0, io1)
    sin = (si0, si1)
    sout = (so0, so1)

    def unit_off(u):
        return (wid * wrows + u * _CH) * D

    def issue_in(u):
        return pltpu.async_copy(
            x_hbm.at[pl.ds(xbase + unit_off(u), nwords)], ios[u % 2], sin[u % 2]
        )

    def issue_out(u):
        return pltpu.async_copy(
            ios[u % 2], o_hbm.at[pl.ds(unit_off(u), nwords)], sout[u % 2]
        )

    pending_out = {}
    pending_in = {0: issue_in(0)}
    for u in range(nunits):
        pltpu.sync_copy(t_hbm.at[pl.ds(unit_off(u), nwords)], t_buf)
        if u + 1 < nunits:
            if u - 1 >= 0:
                pending_out.pop(u - 1).wait()
            pending_in[u + 1] = issue_in(u + 1)
        pending_in.pop(u).wait()
        io = ios[u % 2]

        @plsc.parallel_loop(0, nwords, step=16, unroll=8)
        def _add(k):
            io[pl.ds(k, 16)] = io[pl.ds(k, 16)] + t_buf[pl.ds(k, 16)]

        pending_out[u] = issue_out(u)
    for u in sorted(pending_out):
        pending_out.pop(u).wait()


def _tc_body(x_ref, t_ref, o_ref):
    o_ref[...] = x_ref[...] + t_ref[...][None]


def kernel(inputs, pos_embedding_table):
    B, S, D = inputs.shape
    BS = 512
    tc_out = pl.pallas_call(
        _tc_body,
        grid=(S // BS, B - 1),
        in_specs=[
            pl.BlockSpec((1, BS, D), lambda i, b: (b, i, 0)),
            pl.BlockSpec((BS, D), lambda i, b: (i, 0)),
        ],
        out_specs=pl.BlockSpec((1, BS, D), lambda i, b: (b, i, 0)),
        out_shape=jax.ShapeDtypeStruct((B - 1, S, D), inputs.dtype),
    )(inputs, pos_embedding_table)

    x = inputs.reshape(B * S * D)
    t = pos_embedding_table.reshape(S * D)
    mesh = plsc.VectorSubcoreMesh(core_axis_name="c", subcore_axis_name="s")
    sc_out = pl.kernel(
        _sc_body,
        out_type=jax.ShapeDtypeStruct((S * D,), inputs.dtype),
        mesh=mesh,
        scratch_types=(
            [pltpu.VMEM((_CH * D,), jnp.float32)] * 3
            + [pltpu.SemaphoreType.DMA] * 4
        ),
    )(x, t)
    return jnp.concatenate([tc_out, sc_out.reshape(1, S, D)], axis=0)
